# Initial kernel scaffold; baseline (speedup 1.0000x reference)
import jax
import jax.numpy as jnp
from jax.experimental import pallas as pl


def kernel(inputs, lookup_table):
    raise NotImplementedError("write your pallas kernel here")


# speedup vs baseline: 2.1580x; 2.1580x over previous
import functools

import jax
import jax.numpy as jnp
from jax import lax
from jax.experimental import pallas as pl
from jax.experimental.pallas import tpu as pltpu
from jax.experimental.pallas import tpu_sc as plsc

_info = plsc.get_sparse_core_info()
_NC, _NS = _info.num_cores, _info.num_subcores
_NW = _NC * _NS

_CHUNK = 128
_PITCH = 128


@functools.partial(jax.jit, static_argnums=(2, 3))
def _gather_sc(idx, table_pad, num_chunks, depth):
    mesh = plsc.VectorSubcoreMesh(core_axis_name="c", subcore_axis_name="s")

    @functools.partial(
        pl.kernel,
        mesh=mesh,
        out_type=jax.ShapeDtypeStruct((num_chunks, _CHUNK, _PITCH), jnp.float32),
        scratch_types=[
            pltpu.VMEM((_CHUNK,), jnp.int32),
            pltpu.VMEM((_CHUNK, _PITCH), jnp.float32),
            pltpu.SemaphoreType.DMA,
        ],
    )
    def body(idx_hbm, table_hbm, out_hbm, idx_v, rows_v, sem):
        wid = lax.axis_index("s") * _NC + lax.axis_index("c")
        my_chunks = (num_chunks - wid + _NW - 1) // _NW

        def step(i, _):
            c = wid + i * _NW
            pltpu.sync_copy(idx_hbm.at[pl.ds(c * _CHUNK, _CHUNK)], idx_v)
            pltpu.async_copy(table_hbm.at[idx_v], rows_v, sem).wait()
            pltpu.sync_copy(rows_v, out_hbm.at[c])
            return ()

        lax.fori_loop(0, my_chunks, step, ())

    return body(idx, table_pad)


def kernel(inputs, lookup_table):
    b, n = inputs.shape[0], inputs.shape[1]
    depth = lookup_table.shape[1]
    total = b * n

    idx = inputs.reshape(total)
    num_chunks = -(-total // _CHUNK)
    padded = num_chunks * _CHUNK
    if padded != total:
        idx = jnp.concatenate([idx, jnp.zeros((padded - total,), jnp.int32)])

    table_pad = jnp.pad(lookup_table, ((0, 0), (0, _PITCH - depth)))

    out = _gather_sc(idx, table_pad, num_chunks, depth)
    out = out.reshape(padded, _PITCH)[:, :depth]
    if padded != total:
        out = out[:total]
    return out.reshape(b, n, depth)
